# baseline (speedup 1.0000x reference)
import functools

import jax
import jax.numpy as jnp
from jax import lax
from jax.experimental import pallas as pl
from jax.experimental.pallas import tpu as pltpu
from jax.experimental.pallas import tpu_sc as plsc


_BB = 256


def _argmin_body(q_ref, ktT_ref, idx_ref):
    ktT = ktT_ref[...]
    t2 = jnp.sum(ktT * ktT, axis=0, keepdims=True)
    q = q_ref[...]
    q2 = jnp.sum(q * q, axis=1, keepdims=True)
    d2 = q2 + t2 - 2.0 * jnp.dot(q, ktT, preferred_element_type=jnp.float32)
    dist = jnp.sqrt(jnp.maximum(d2, 0.0))
    m = jnp.min(dist, axis=1, keepdims=True)
    k = dist.shape[1]
    iota = lax.broadcasted_iota(jnp.int32, dist.shape, 1)
    idx_ref[...] = jnp.min(jnp.where(dist == m, iota, k), axis=1).astype(jnp.int32)


def _nearest_idx(q, ktT):
    b, dk = q.shape
    k = ktT.shape[1]
    return pl.pallas_call(
        _argmin_body,
        grid=(b // _BB,),
        in_specs=[
            pl.BlockSpec((_BB, dk), lambda i: (i, 0)),
            pl.BlockSpec((dk, k), lambda i: (0, 0)),
        ],
        out_specs=pl.BlockSpec((_BB,), lambda i: (i,)),
        out_shape=jax.ShapeDtypeStruct((b,), jnp.int32),
    )(q, ktT)


_NC, _NS, _L = 2, 16, 16
_NW = _NC * _NS


def _sc_residual(table, idx, feat_flat, fidx):
    kk, c_total = table.shape
    b = idx.shape[0]
    df = fidx.shape[0]
    bpw = b // _NW
    nch = df // _L
    mesh = plsc.VectorSubcoreMesh(
        core_axis_name="c", subcore_axis_name="s",
        num_cores=_NC, num_subcores=_NS)

    @functools.partial(
        pl.kernel,
        out_type=jax.ShapeDtypeStruct((b * df,), jnp.float32),
        mesh=mesh,
        scratch_types=[
            pltpu.VMEM((bpw,), jnp.int32),
            pltpu.VMEM((bpw, c_total), jnp.float32),
            pltpu.VMEM((bpw * df,), jnp.float32),
            pltpu.VMEM((df,), jnp.int32),
            pltpu.SemaphoreType.DMA,
        ],
        compiler_params=pltpu.CompilerParams(
            use_tc_tiling_on_sc=False, needs_layout_passes=False),
    )
    def body(table_hbm, idx_hbm, feat_hbm, fidx_hbm, out_hbm,
             idx_v, rows_v, feat_v, fidx_v, sem):
        wid = lax.axis_index("s") * _NC + lax.axis_index("c")
        base = wid * bpw
        pltpu.sync_copy(idx_hbm.at[pl.ds(base, bpw)], idx_v)
        cp = pltpu.async_copy(table_hbm.at[idx_v], rows_v, sem)
        pltpu.sync_copy(feat_hbm.at[pl.ds(base * df, bpw * df)], feat_v)
        pltpu.sync_copy(fidx_hbm, fidx_v)
        cp.wait()
        cols = [fidx_v[pl.ds(c * _L, _L)] for c in range(nch)]

        def row_body(r, carry):
            rsplat = jnp.full((_L,), r, jnp.int32)
            for c in range(nch):
                vals = plsc.load_gather(rows_v, [rsplat, cols[c]])
                o = pl.multiple_of(r * df + c * _L, _L)
                feat_v[pl.ds(o, _L)] = feat_v[pl.ds(o, _L)] - vals
            return carry

        lax.fori_loop(0, bpw, row_body, 0)
        pltpu.sync_copy(feat_v, out_hbm.at[pl.ds(base * df, bpw * df)])

    return body(table, idx, feat_flat, fidx)


def kernel(predicted_key, features, lookup_table, lookup_key_indices,
           feature_indices):
    b, df = features.shape
    key_table = jnp.take(lookup_table, lookup_key_indices, axis=1)
    idx = _nearest_idx(predicted_key, key_table.T)
    out_flat = _sc_residual(lookup_table, idx, features.reshape(-1),
                            feature_indices)
    return out_flat.reshape(b, df)


# speedup vs baseline: 1.0126x; 1.0126x over previous
import functools

import jax
import jax.numpy as jnp
from jax import lax
from jax.experimental import pallas as pl
from jax.experimental.pallas import tpu as pltpu
from jax.experimental.pallas import tpu_sc as plsc


_BB = 256


def _argmin_body(q_ref, ktT_ref, idx_ref, t2_ref):
    @pl.when(pl.program_id(0) == 0)
    def _():
        ktT = ktT_ref[...]
        t2_ref[...] = jnp.sum(ktT * ktT, axis=0, keepdims=True)

    q = q_ref[...]
    q2 = jnp.sum(q * q, axis=1, keepdims=True)
    qt = jnp.dot(q, ktT_ref[...], preferred_element_type=jnp.float32)
    d2 = q2 + t2_ref[...] - 2.0 * qt
    dist = jnp.sqrt(jnp.maximum(d2, 0.0))
    m = jnp.min(dist, axis=1, keepdims=True)
    k = dist.shape[1]
    iota = lax.broadcasted_iota(jnp.int32, dist.shape, 1)
    idx_ref[...] = jnp.min(jnp.where(dist == m, iota, k), axis=1).astype(jnp.int32)


def _nearest_idx(q, ktT):
    b, dk = q.shape
    k = ktT.shape[1]
    return pl.pallas_call(
        _argmin_body,
        grid=(b // _BB,),
        in_specs=[
            pl.BlockSpec((_BB, dk), lambda i: (i, 0)),
            pl.BlockSpec((dk, k), lambda i: (0, 0)),
        ],
        out_specs=pl.BlockSpec((_BB,), lambda i: (i,)),
        out_shape=jax.ShapeDtypeStruct((b,), jnp.int32),
        scratch_shapes=[pltpu.VMEM((1, k), jnp.float32)],
    )(q, ktT)


_NC, _NS, _L = 2, 16, 16
_NW = _NC * _NS


def _sc_residual(table, idx, feat, fidx):
    kk, c_total = table.shape
    b = idx.shape[0]
    df = fidx.shape[0]
    bpw = b // _NW
    nch = df // _L
    mesh = plsc.VectorSubcoreMesh(
        core_axis_name="c", subcore_axis_name="s",
        num_cores=_NC, num_subcores=_NS)

    @functools.partial(
        pl.kernel,
        out_type=jax.ShapeDtypeStruct((b, df), jnp.float32),
        mesh=mesh,
        scratch_types=[
            pltpu.VMEM((bpw,), jnp.int32),
            pltpu.VMEM((bpw, c_total), jnp.float32),
            pltpu.VMEM((bpw, df), jnp.float32),
            pltpu.VMEM((df,), jnp.int32),
            pltpu.SemaphoreType.DMA,
        ],
        compiler_params=pltpu.CompilerParams(
            use_tc_tiling_on_sc=False, needs_layout_passes=False),
    )
    def body(table_hbm, idx_hbm, feat_hbm, fidx_hbm, out_hbm,
             idx_v, rows_v, feat_v, fidx_v, sem):
        wid = lax.axis_index("s") * _NC + lax.axis_index("c")
        base = wid * bpw
        pltpu.sync_copy(idx_hbm.at[pl.ds(base, bpw)], idx_v)
        cp = pltpu.async_copy(table_hbm.at[idx_v], rows_v, sem)
        pltpu.sync_copy(feat_hbm.at[pl.ds(base, bpw)], feat_v)
        pltpu.sync_copy(fidx_hbm, fidx_v)
        cp.wait()
        cols = [fidx_v[pl.ds(c * _L, _L)] for c in range(nch)]

        def row_body(r, carry):
            rsplat = jnp.full((_L,), r, jnp.int32)
            for c in range(nch):
                vals = plsc.load_gather(rows_v, [rsplat, cols[c]])
                feat_v[r, pl.ds(c * _L, _L)] = feat_v[r, pl.ds(c * _L, _L)] - vals
            return carry

        lax.fori_loop(0, bpw, row_body, 0)
        pltpu.sync_copy(feat_v, out_hbm.at[pl.ds(base, bpw)])

    return body(table, idx, feat, fidx)


def kernel(predicted_key, features, lookup_table, lookup_key_indices,
           feature_indices):
    key_table = jnp.take(lookup_table, lookup_key_indices, axis=1)
    idx = _nearest_idx(predicted_key, key_table.T)
    return _sc_residual(lookup_table, idx, features, feature_indices)


# speedup vs baseline: 1.2027x; 1.1878x over previous
import functools

import jax
import jax.numpy as jnp
from jax import lax
from jax.experimental import pallas as pl
from jax.experimental.pallas import tpu as pltpu
from jax.experimental.pallas import tpu_sc as plsc


_BB = 256


def _argmin_body(q_ref, ktT_ref, t2_ref, q2_ref, idx_ref, d2s_ref, iota_ref):
    @pl.when(pl.program_id(0) == 0)
    def _():
        iota_ref[...] = lax.broadcasted_iota(jnp.int32, iota_ref.shape, 1)

    q2 = q2_ref[...]
    qt = jnp.dot(q_ref[...], ktT_ref[...], preferred_element_type=jnp.float32)
    d2 = (q2 + t2_ref[...]) - 2.0 * qt
    d2s_ref[...] = d2
    m2 = jnp.min(d2, axis=1, keepdims=True)
    s = jnp.sqrt(jnp.maximum(m2, 0.0))
    yb = lax.bitcast_convert_type(s * s, jnp.int32)
    hi = jnp.full_like(s, -jnp.inf)
    for koff in range(-4, 5):
        xk = lax.bitcast_convert_type(jnp.maximum(yb + koff, 0), jnp.float32)
        ok = jnp.sqrt(jnp.maximum(xk, 0.0)) == s
        hi = jnp.maximum(hi, jnp.where(ok, xk, -jnp.inf))
    hi = jnp.where(s == 0.0, 0.0, hi)
    k = d2.shape[1]
    cand = jnp.where(d2s_ref[...] <= hi, iota_ref[...], k)
    idx_ref[...] = jnp.min(cand, axis=1).astype(jnp.int32)


def _nearest_idx(q, ktT, t2, q2):
    b, dk = q.shape
    k = ktT.shape[1]
    return pl.pallas_call(
        _argmin_body,
        grid=(b // _BB,),
        in_specs=[
            pl.BlockSpec((_BB, dk), lambda i: (i, 0)),
            pl.BlockSpec((dk, k), lambda i: (0, 0)),
            pl.BlockSpec((1, k), lambda i: (0, 0)),
            pl.BlockSpec((_BB, 1), lambda i: (i, 0)),
        ],
        out_specs=pl.BlockSpec((_BB,), lambda i: (i,)),
        out_shape=jax.ShapeDtypeStruct((b,), jnp.int32),
        scratch_shapes=[
            pltpu.VMEM((_BB, k), jnp.float32),
            pltpu.VMEM((_BB, k), jnp.int32),
        ],
    )(q, ktT, t2, q2)


_NC, _NS, _L = 2, 16, 16
_NW = _NC * _NS


def _sc_residual(table, idx, feat, fidx):
    kk, c_total = table.shape
    b = idx.shape[0]
    df = fidx.shape[0]
    bpw = b // _NW
    nch = df // _L
    mesh = plsc.VectorSubcoreMesh(
        core_axis_name="c", subcore_axis_name="s",
        num_cores=_NC, num_subcores=_NS)

    @functools.partial(
        pl.kernel,
        out_type=jax.ShapeDtypeStruct((b, df), jnp.float32),
        mesh=mesh,
        scratch_types=[
            pltpu.VMEM((bpw,), jnp.int32),
            pltpu.VMEM((bpw, c_total), jnp.float32),
            pltpu.VMEM((bpw, df), jnp.float32),
            pltpu.VMEM((df,), jnp.int32),
            pltpu.SemaphoreType.DMA,
        ],
        compiler_params=pltpu.CompilerParams(
            use_tc_tiling_on_sc=False, needs_layout_passes=False),
    )
    def body(table_hbm, idx_hbm, feat_hbm, fidx_hbm, out_hbm,
             idx_v, rows_v, feat_v, fidx_v, sem):
        wid = lax.axis_index("s") * _NC + lax.axis_index("c")
        base = wid * bpw
        pltpu.sync_copy(idx_hbm.at[pl.ds(base, bpw)], idx_v)
        cp = pltpu.async_copy(table_hbm.at[idx_v], rows_v, sem)
        pltpu.sync_copy(feat_hbm.at[pl.ds(base, bpw)], feat_v)
        pltpu.sync_copy(fidx_hbm, fidx_v)
        cp.wait()
        cols = [fidx_v[pl.ds(c * _L, _L)] for c in range(nch)]

        def row_body(r, carry):
            rsplat = jnp.full((_L,), r, jnp.int32)
            for c in range(nch):
                vals = plsc.load_gather(rows_v, [rsplat, cols[c]])
                feat_v[r, pl.ds(c * _L, _L)] = feat_v[r, pl.ds(c * _L, _L)] - vals
            return carry

        lax.fori_loop(0, bpw, row_body, 0)
        pltpu.sync_copy(feat_v, out_hbm.at[pl.ds(base, bpw)])

    return body(table, idx, feat, fidx)


def kernel(predicted_key, features, lookup_table, lookup_key_indices,
           feature_indices):
    key_table = jnp.take(lookup_table, lookup_key_indices, axis=1)
    t2 = jnp.sum(key_table * key_table, axis=1)[None, :]
    q2 = jnp.sum(predicted_key * predicted_key, axis=1, keepdims=True)
    idx = _nearest_idx(predicted_key, key_table.T, t2, q2)
    return _sc_residual(lookup_table, idx, features, feature_indices)


# speedup vs baseline: 1.2141x; 1.0094x over previous
import functools

import jax
import jax.numpy as jnp
from jax import lax
from jax.experimental import pallas as pl
from jax.experimental.pallas import tpu as pltpu
from jax.experimental.pallas import tpu_sc as plsc


_BB = 256


def _argmin_body(q_ref, ktT_ref, t2_ref, q2_ref, idx_ref, d2s_ref, iota_ref):
    @pl.when(pl.program_id(0) == 0)
    def _():
        iota_ref[...] = lax.broadcasted_iota(jnp.int32, iota_ref.shape, 1)

    q2 = q2_ref[...]
    qt = jnp.dot(q_ref[...], ktT_ref[...], preferred_element_type=jnp.float32)
    d2 = (q2 + t2_ref[...]) - 2.0 * qt
    d2s_ref[...] = d2
    m2 = jnp.min(d2, axis=1, keepdims=True)
    s = jnp.sqrt(jnp.maximum(m2, 0.0))
    yb = lax.bitcast_convert_type(s * s, jnp.int32)
    hi = jnp.full_like(s, -jnp.inf)
    for koff in range(-4, 5):
        xk = lax.bitcast_convert_type(jnp.maximum(yb + koff, 0), jnp.float32)
        ok = jnp.sqrt(jnp.maximum(xk, 0.0)) == s
        hi = jnp.maximum(hi, jnp.where(ok, xk, -jnp.inf))
    hi = jnp.where(s == 0.0, 0.0, hi)
    k = d2.shape[1]
    cand = jnp.where(d2s_ref[...] <= hi, iota_ref[...], k)
    idx_ref[...] = jnp.min(cand, axis=1).astype(jnp.int32)


def _nearest_idx(q, ktT, t2, q2):
    b, dk = q.shape
    k = ktT.shape[1]
    return pl.pallas_call(
        _argmin_body,
        grid=(b // _BB,),
        in_specs=[
            pl.BlockSpec((_BB, dk), lambda i: (i, 0)),
            pl.BlockSpec((dk, k), lambda i: (0, 0)),
            pl.BlockSpec((1, k), lambda i: (0, 0)),
            pl.BlockSpec((_BB, 1), lambda i: (i, 0)),
        ],
        out_specs=pl.BlockSpec((_BB,), lambda i: (i,)),
        out_shape=jax.ShapeDtypeStruct((b,), jnp.int32),
        scratch_shapes=[
            pltpu.VMEM((_BB, k), jnp.float32),
            pltpu.VMEM((_BB, k), jnp.int32),
        ],
    )(q, ktT, t2, q2)


_NC, _NS, _L = 2, 16, 16
_NW = _NC * _NS


def _sc_residual(table, idx, feat, fidx):
    kk, c_total = table.shape
    b = idx.shape[0]
    df = fidx.shape[0]
    bpw = b // _NW
    nch = df // _L
    mesh = plsc.VectorSubcoreMesh(
        core_axis_name="c", subcore_axis_name="s",
        num_cores=_NC, num_subcores=_NS)

    @functools.partial(
        pl.kernel,
        out_type=jax.ShapeDtypeStruct((b, df), jnp.float32),
        mesh=mesh,
        scratch_types=[
            pltpu.VMEM((bpw,), jnp.int32),
            pltpu.VMEM((bpw, c_total), jnp.float32),
            pltpu.VMEM((bpw, df), jnp.float32),
            pltpu.VMEM((df,), jnp.int32),
            pltpu.SemaphoreType.DMA,
        ],
        compiler_params=pltpu.CompilerParams(
            use_tc_tiling_on_sc=False, needs_layout_passes=False),
    )
    def body(table_hbm, idx_hbm, feat_hbm, fidx_hbm, out_hbm,
             idx_v, rows_v, feat_v, fidx_v, sem):
        wid = lax.axis_index("s") * _NC + lax.axis_index("c")
        base = wid * bpw
        pltpu.sync_copy(idx_hbm.at[pl.ds(base, bpw)], idx_v)
        cp = pltpu.async_copy(table_hbm.at[idx_v], rows_v, sem)
        pltpu.sync_copy(feat_hbm.at[pl.ds(base, bpw)], feat_v)
        pltpu.sync_copy(fidx_hbm, fidx_v)
        cp.wait()
        cols = [fidx_v[pl.ds(c * _L, _L)] for c in range(nch)]

        def row_body(r, carry):
            rsplat = jnp.full((_L,), r, jnp.int32)
            for c in range(nch):
                vals = plsc.load_gather(rows_v, [rsplat, cols[c]])
                feat_v[r, pl.ds(c * _L, _L)] = feat_v[r, pl.ds(c * _L, _L)] - vals
            return carry

        lax.fori_loop(0, bpw, row_body, 0)
        pltpu.sync_copy(feat_v, out_hbm.at[pl.ds(base, bpw)])

    return body(table, idx, feat, fidx)


_NCHUNKS = 2


def kernel(predicted_key, features, lookup_table, lookup_key_indices,
           feature_indices):
    b = predicted_key.shape[0]
    key_table = jnp.take(lookup_table, lookup_key_indices, axis=1)
    t2 = jnp.sum(key_table * key_table, axis=1)[None, :]
    q2 = jnp.sum(predicted_key * predicted_key, axis=1, keepdims=True)
    ktT = key_table.T
    bc = b // _NCHUNKS
    outs = []
    for c in range(_NCHUNKS):
        sl = slice(c * bc, (c + 1) * bc)
        idx_c = _nearest_idx(predicted_key[sl], ktT, t2, q2[sl])
        outs.append(_sc_residual(lookup_table, idx_c, features[sl],
                                 feature_indices))
    return jnp.concatenate(outs, axis=0)
